# Initial kernel scaffold; baseline (speedup 1.0000x reference)
import jax
import jax.numpy as jnp
from jax.experimental import pallas as pl


def kernel(x, edge_index, W1, b1, W2, b2):
    raise NotImplementedError("write your pallas kernel here")


# speedup vs baseline: 31.5209x; 31.5209x over previous
import functools

import jax
import jax.numpy as jnp
from jax import lax
from jax.experimental import pallas as pl
from jax.experimental.pallas import tpu as pltpu
from jax.experimental.pallas import tpu_sc as plsc

N = 10000
NP = 10240
E = 320000
NC, NS = 2, 16
NW = NC * NS
C = 128
CHUNKS = 80
EP = NW * CHUNKS * C
W = 8

f32 = jnp.float32


def _mesh():
    return plsc.VectorSubcoreMesh(
        core_axis_name="c", subcore_axis_name="s", num_cores=NC, num_subcores=NS
    )


_SC_PARAMS = pltpu.CompilerParams(use_tc_tiling_on_sc=False)


@functools.partial(
    pl.kernel,
    mesh=_mesh(),
    out_type=jax.ShapeDtypeStruct((NC, NP, W), f32),
    compiler_params=_SC_PARAMS,
    scratch_types=[
        pltpu.VMEM((CHUNKS, C), jnp.int32),
        pltpu.VMEM((C, W), f32),
        pltpu.VMEM_SHARED((NP, W), f32),
    ],
)
def _deg_kernel(dst_hbm, zeros_hbm, ones_hbm, out_hbm, idx_v, ones_v, acc_sh):
    cid = lax.axis_index("c")
    sid = lax.axis_index("s")
    wid = sid * NC + cid

    @pl.when(sid == 0)
    def _():
        pltpu.sync_copy(zeros_hbm, acc_sh)

    pltpu.sync_copy(ones_hbm, ones_v)
    pltpu.sync_copy(dst_hbm.at[wid], idx_v)
    plsc.subcore_barrier()

    @pl.loop(0, CHUNKS)
    def _(j):
        pltpu.sync_copy(ones_v, acc_sh.at[idx_v.at[j]], add=True)

    plsc.subcore_barrier()

    @pl.when(sid == 0)
    def _():
        pltpu.sync_copy(acc_sh, out_hbm.at[cid])


@functools.partial(
    pl.kernel,
    mesh=_mesh(),
    out_type=jax.ShapeDtypeStruct((NC, NP, W), f32),
    compiler_params=_SC_PARAMS,
    scratch_types=[
        pltpu.VMEM((CHUNKS, C), jnp.int32),
        pltpu.VMEM((CHUNKS, C), jnp.int32),
        pltpu.VMEM((C, W), f32),
        pltpu.VMEM_SHARED((NP, W), f32),
    ],
)
def _scat_kernel(src_hbm, dst_hbm, g_hbm, zeros_hbm, out_hbm,
                 src_v, dst_v, msg_v, acc_sh):
    cid = lax.axis_index("c")
    sid = lax.axis_index("s")
    wid = sid * NC + cid

    @pl.when(sid == 0)
    def _():
        pltpu.sync_copy(zeros_hbm, acc_sh)

    pltpu.sync_copy(src_hbm.at[wid], src_v)
    pltpu.sync_copy(dst_hbm.at[wid], dst_v)
    plsc.subcore_barrier()

    @pl.loop(0, CHUNKS)
    def _(j):
        pltpu.sync_copy(g_hbm.at[src_v.at[j]], msg_v)
        pltpu.sync_copy(msg_v, acc_sh.at[dst_v.at[j]], add=True)

    plsc.subcore_barrier()

    @pl.when(sid == 0)
    def _():
        pltpu.sync_copy(acc_sh, out_hbm.at[cid])


def _tc1(xp, W1, deg):
    def body(x_ref, w_ref, deg_ref, dis_ref, g_ref):
        h = jnp.dot(x_ref[...], w_ref[...], preferred_element_type=f32)
        dis = lax.rsqrt(deg_ref[0][:, :1] + deg_ref[1][:, :1] + 1.0)
        dis_ref[...] = dis
        g_ref[...] = jnp.concatenate([h * dis, jnp.zeros((NP, W - 4), f32)], axis=1)

    return pl.pallas_call(
        body,
        out_shape=(
            jax.ShapeDtypeStruct((NP, 1), f32),
            jax.ShapeDtypeStruct((NP, W), f32),
        ),
    )(xp, W1, deg)


def _tc2(dis, g1, S1, b1, W2):
    def body(dis_ref, g_ref, s_ref, b_ref, w_ref, g2_ref):
        acc = (g_ref[:, :4] + s_ref[0][:, :4] + s_ref[1][:, :4])
        act = jnp.maximum(dis_ref[...] * acc + b_ref[...], 0.0)
        h2 = jnp.dot(act, w_ref[...], preferred_element_type=f32)
        g2_ref[...] = jnp.concatenate(
            [h2 * dis_ref[...], jnp.zeros((NP, W - 2), f32)], axis=1)

    return pl.pallas_call(
        body,
        out_shape=jax.ShapeDtypeStruct((NP, W), f32),
    )(dis, g1, S1, b1, W2)


def _tc3(dis, g2, S2, b2):
    def body(dis_ref, g_ref, s_ref, b_ref, out_ref):
        acc = g_ref[:, :2] + s_ref[0][:, :2] + s_ref[1][:, :2]
        out_ref[...] = dis_ref[...] * acc + b_ref[...]

    return pl.pallas_call(
        body,
        out_shape=jax.ShapeDtypeStruct((NP, 2), f32),
    )(dis, g2, S2, b2)


def kernel(x, edge_index, W1, b1, W2, b2):
    ei = edge_index.astype(jnp.int32)
    pad = jnp.full((EP - E,), N, jnp.int32)
    src = jnp.concatenate([ei[0], pad]).reshape(NW, CHUNKS, C)
    dst = jnp.concatenate([ei[1], pad]).reshape(NW, CHUNKS, C)
    xp = jnp.pad(x, ((0, NP - N), (0, 0)))

    zeros_w = jnp.zeros((NP, W), f32)
    ones_c = jnp.ones((C, W), f32)
    deg = _deg_kernel(dst, zeros_w, ones_c)

    dis, g1 = _tc1(xp, W1, deg)
    S1 = _scat_kernel(src, dst, g1, zeros_w)
    g2 = _tc2(dis, g1, S1, b1.reshape(1, 4), W2)
    S2 = _scat_kernel(src, dst, g2, zeros_w)
    out = _tc3(dis, g2, S2, b2.reshape(1, 2))
    return out[:N]


# speedup vs baseline: 38.8463x; 1.2324x over previous
import functools

import jax
import jax.numpy as jnp
from jax import lax
from jax.experimental import pallas as pl
from jax.experimental.pallas import tpu as pltpu
from jax.experimental.pallas import tpu_sc as plsc

N = 10000
NP = 10240
E = 320000
NC, NS = 2, 16
NW = NC * NS
PT = 10240
EP = NW * PT
W = 8
DEG_C = 2048

f32 = jnp.float32


def _mesh():
    return plsc.VectorSubcoreMesh(
        core_axis_name="c", subcore_axis_name="s", num_cores=NC, num_subcores=NS
    )


_SC_PARAMS = pltpu.CompilerParams(use_tc_tiling_on_sc=False)


@functools.partial(
    pl.kernel,
    mesh=_mesh(),
    out_type=jax.ShapeDtypeStruct((NC, NP, W), f32),
    compiler_params=_SC_PARAMS,
    scratch_types=[
        pltpu.VMEM((PT // DEG_C, DEG_C), jnp.int32),
        pltpu.VMEM((DEG_C, W), f32),
        pltpu.VMEM_SHARED((NP, W), f32),
        pltpu.SemaphoreType.DMA,
    ],
)
def _deg_kernel(dst_hbm, zeros_hbm, ones_hbm, out_hbm, idx_v, ones_v, acc_sh, sem):
    cid = lax.axis_index("c")
    sid = lax.axis_index("s")
    wid = sid * NC + cid

    @pl.when(sid == 0)
    def _():
        pltpu.sync_copy(zeros_hbm, acc_sh)

    pltpu.sync_copy(ones_hbm, ones_v)
    pltpu.sync_copy(dst_hbm.at[wid], idx_v)
    plsc.subcore_barrier()

    for k in range(PT // DEG_C):
        pltpu.async_copy(ones_v, acc_sh.at[idx_v.at[k]], sem, add=True)
    for k in range(PT // DEG_C):
        pltpu.make_async_copy(ones_v, acc_sh.at[idx_v.at[k]], sem).wait()

    plsc.subcore_barrier()

    @pl.when(sid == 0)
    def _():
        pltpu.sync_copy(acc_sh, out_hbm.at[cid])


@functools.partial(
    pl.kernel,
    mesh=_mesh(),
    out_type=jax.ShapeDtypeStruct((NC, NP, W), f32),
    compiler_params=_SC_PARAMS,
    scratch_types=[
        pltpu.VMEM((PT,), jnp.int32),
        pltpu.VMEM((PT,), jnp.int32),
        pltpu.VMEM((PT, W), f32),
        pltpu.VMEM_SHARED((NP, W), f32),
    ],
)
def _scat_kernel(src_hbm, dst_hbm, g_hbm, zeros_hbm, out_hbm,
                 src_v, dst_v, msg_v, acc_sh):
    cid = lax.axis_index("c")
    sid = lax.axis_index("s")
    wid = sid * NC + cid

    @pl.when(sid == 0)
    def _():
        pltpu.sync_copy(zeros_hbm, acc_sh)

    pltpu.sync_copy(src_hbm.at[wid], src_v)
    pltpu.sync_copy(dst_hbm.at[wid], dst_v)
    plsc.subcore_barrier()

    pltpu.sync_copy(g_hbm.at[src_v], msg_v)
    pltpu.sync_copy(msg_v, acc_sh.at[dst_v], add=True)

    plsc.subcore_barrier()

    @pl.when(sid == 0)
    def _():
        pltpu.sync_copy(acc_sh, out_hbm.at[cid])


def _tc1(xp, W1, deg):
    def body(x_ref, w_ref, deg_ref, dis_ref, g_ref):
        h = jnp.dot(x_ref[...], w_ref[...], preferred_element_type=f32)
        dis = lax.rsqrt(deg_ref[0][:, :1] + deg_ref[1][:, :1] + 1.0)
        dis_ref[...] = dis
        g_ref[...] = jnp.concatenate([h * dis, jnp.zeros((NP, W - 4), f32)], axis=1)

    return pl.pallas_call(
        body,
        out_shape=(
            jax.ShapeDtypeStruct((NP, 1), f32),
            jax.ShapeDtypeStruct((NP, W), f32),
        ),
    )(xp, W1, deg)


def _tc2(dis, g1, S1, b1, W2):
    def body(dis_ref, g_ref, s_ref, b_ref, w_ref, g2_ref):
        acc = (g_ref[:, :4] + s_ref[0][:, :4] + s_ref[1][:, :4])
        act = jnp.maximum(dis_ref[...] * acc + b_ref[...], 0.0)
        h2 = jnp.dot(act, w_ref[...], preferred_element_type=f32)
        g2_ref[...] = jnp.concatenate(
            [h2 * dis_ref[...], jnp.zeros((NP, W - 2), f32)], axis=1)

    return pl.pallas_call(
        body,
        out_shape=jax.ShapeDtypeStruct((NP, W), f32),
    )(dis, g1, S1, b1, W2)


def _tc3(dis, g2, S2, b2):
    def body(dis_ref, g_ref, s_ref, b_ref, out_ref):
        acc = g_ref[:, :2] + s_ref[0][:, :2] + s_ref[1][:, :2]
        out_ref[...] = dis_ref[...] * acc + b_ref[...]

    return pl.pallas_call(
        body,
        out_shape=jax.ShapeDtypeStruct((NP, 2), f32),
    )(dis, g2, S2, b2)


def kernel(x, edge_index, W1, b1, W2, b2):
    ei = edge_index.astype(jnp.int32)
    pad = jnp.full((EP - E,), N, jnp.int32)
    src = jnp.concatenate([ei[0], pad]).reshape(NW, PT)
    dst = jnp.concatenate([ei[1], pad]).reshape(NW, PT)
    xp = jnp.pad(x, ((0, NP - N), (0, 0)))

    zeros_w = jnp.zeros((NP, W), f32)
    ones_c = jnp.ones((DEG_C, W), f32)
    deg = _deg_kernel(dst.reshape(NW, PT // DEG_C, DEG_C), zeros_w, ones_c)

    dis, g1 = _tc1(xp, W1, deg)
    S1 = _scat_kernel(src, dst, g1, zeros_w)
    g2 = _tc2(dis, g1, S1, b1.reshape(1, 4), W2)
    S2 = _scat_kernel(src, dst, g2, zeros_w)
    out = _tc3(dis, g2, S2, b2.reshape(1, 2))
    return out[:N]


# speedup vs baseline: 40.7125x; 1.0480x over previous
import functools

import jax
import jax.numpy as jnp
from jax import lax
from jax.experimental import pallas as pl
from jax.experimental.pallas import tpu as pltpu
from jax.experimental.pallas import tpu_sc as plsc

N = 10000
NP = 10240
E = 320000
NC, NS = 2, 16
NW = NC * NS
PT = 10240
EP = NW * PT
W = 8
DEG_C = 2048

f32 = jnp.float32


def _mesh():
    return plsc.VectorSubcoreMesh(
        core_axis_name="c", subcore_axis_name="s", num_cores=NC, num_subcores=NS
    )


_SC_PARAMS = pltpu.CompilerParams(use_tc_tiling_on_sc=False)


@functools.partial(
    pl.kernel,
    mesh=_mesh(),
    out_type=jax.ShapeDtypeStruct((NC, NP, W), f32),
    compiler_params=_SC_PARAMS,
    scratch_types=[
        pltpu.VMEM((PT // DEG_C, DEG_C), jnp.int32),
        pltpu.VMEM((DEG_C, W), f32),
        pltpu.VMEM_SHARED((NP, W), f32),
        pltpu.SemaphoreType.DMA,
    ],
)
def _deg_kernel(dst_hbm, zeros_hbm, ones_hbm, out_hbm, idx_v, ones_v, acc_sh, sem):
    cid = lax.axis_index("c")
    sid = lax.axis_index("s")
    wid = sid * NC + cid

    @pl.when(sid == 0)
    def _():
        pltpu.sync_copy(zeros_hbm, acc_sh)

    pltpu.sync_copy(ones_hbm, ones_v)
    pltpu.sync_copy(dst_hbm.at[wid], idx_v)
    plsc.subcore_barrier()

    for k in range(PT // DEG_C):
        pltpu.async_copy(ones_v, acc_sh.at[idx_v.at[k]], sem, add=True)
    for k in range(PT // DEG_C):
        pltpu.make_async_copy(ones_v, acc_sh.at[idx_v.at[k]], sem).wait()

    plsc.subcore_barrier()

    @pl.when(sid == 0)
    def _():
        pltpu.sync_copy(acc_sh, out_hbm.at[cid])


CH = 8
CL = PT // CH


@functools.partial(
    pl.kernel,
    mesh=_mesh(),
    out_type=jax.ShapeDtypeStruct((NC, NP, W), f32),
    compiler_params=_SC_PARAMS,
    scratch_types=[
        pltpu.VMEM((CH, CL), jnp.int32),
        pltpu.VMEM((CH, CL), jnp.int32),
        pltpu.VMEM((2, CL, W), f32),
        pltpu.VMEM_SHARED((NP, W), f32),
        pltpu.SemaphoreType.DMA,
        pltpu.SemaphoreType.DMA,
        pltpu.SemaphoreType.DMA,
        pltpu.SemaphoreType.DMA,
    ],
)
def _scat_kernel(src_hbm, dst_hbm, g_hbm, zeros_hbm, out_hbm,
                 src_v, dst_v, msg_v, acc_sh, gs0, gs1, ss0, ss1):
    cid = lax.axis_index("c")
    sid = lax.axis_index("s")
    wid = sid * NC + cid
    gsem = (gs0, gs1)
    ssem = (ss0, ss1)

    @pl.when(sid == 0)
    def _():
        pltpu.sync_copy(zeros_hbm, acc_sh)

    pltpu.sync_copy(src_hbm.at[wid], src_v)
    pltpu.sync_copy(dst_hbm.at[wid], dst_v)
    plsc.subcore_barrier()

    def gather(c, b):
        return pltpu.async_copy(g_hbm.at[src_v.at[c]], msg_v.at[b], gsem[b])

    def scatter(c, b):
        return pltpu.async_copy(msg_v.at[b], acc_sh.at[dst_v.at[c]], ssem[b],
                                add=True)

    gather(0, 0)
    for c in range(CH):
        b = c % 2
        pltpu.make_async_copy(g_hbm.at[src_v.at[c]], msg_v.at[b], gsem[b]).wait()
        scatter(c, b)
        if c + 1 < CH:
            nb = 1 - b
            if c >= 1:
                pltpu.make_async_copy(
                    msg_v.at[nb], acc_sh.at[dst_v.at[c - 1]], ssem[nb]).wait()
            gather(c + 1, nb)
    pltpu.make_async_copy(
        msg_v.at[(CH - 1) % 2], acc_sh.at[dst_v.at[CH - 1]],
        ssem[(CH - 1) % 2]).wait()
    pltpu.make_async_copy(
        msg_v.at[(CH - 2) % 2], acc_sh.at[dst_v.at[CH - 2]],
        ssem[(CH - 2) % 2]).wait()

    plsc.subcore_barrier()

    @pl.when(sid == 0)
    def _():
        pltpu.sync_copy(acc_sh, out_hbm.at[cid])


def _tc1(xp, W1, deg):
    def body(x_ref, w_ref, deg_ref, dis_ref, g_ref):
        h = jnp.dot(x_ref[...], w_ref[...], preferred_element_type=f32)
        dis = lax.rsqrt(deg_ref[0][:, :1] + deg_ref[1][:, :1] + 1.0)
        dis_ref[...] = dis
        g_ref[...] = jnp.concatenate([h * dis, jnp.zeros((NP, W - 4), f32)], axis=1)

    return pl.pallas_call(
        body,
        out_shape=(
            jax.ShapeDtypeStruct((NP, 1), f32),
            jax.ShapeDtypeStruct((NP, W), f32),
        ),
    )(xp, W1, deg)


def _tc2(dis, g1, S1, b1, W2):
    def body(dis_ref, g_ref, s_ref, b_ref, w_ref, g2_ref):
        acc = (g_ref[:, :4] + s_ref[0][:, :4] + s_ref[1][:, :4])
        act = jnp.maximum(dis_ref[...] * acc + b_ref[...], 0.0)
        h2 = jnp.dot(act, w_ref[...], preferred_element_type=f32)
        g2_ref[...] = jnp.concatenate(
            [h2 * dis_ref[...], jnp.zeros((NP, W - 2), f32)], axis=1)

    return pl.pallas_call(
        body,
        out_shape=jax.ShapeDtypeStruct((NP, W), f32),
    )(dis, g1, S1, b1, W2)


def _tc3(dis, g2, S2, b2):
    def body(dis_ref, g_ref, s_ref, b_ref, out_ref):
        acc = g_ref[:, :2] + s_ref[0][:, :2] + s_ref[1][:, :2]
        out_ref[...] = dis_ref[...] * acc + b_ref[...]

    return pl.pallas_call(
        body,
        out_shape=jax.ShapeDtypeStruct((NP, 2), f32),
    )(dis, g2, S2, b2)


def kernel(x, edge_index, W1, b1, W2, b2):
    ei = edge_index.astype(jnp.int32)
    pad = jnp.full((EP - E,), N, jnp.int32)
    src = jnp.concatenate([ei[0], pad]).reshape(NW, PT)
    dst = jnp.concatenate([ei[1], pad]).reshape(NW, PT)
    xp = jnp.pad(x, ((0, NP - N), (0, 0)))

    zeros_w = jnp.zeros((NP, W), f32)
    ones_c = jnp.ones((DEG_C, W), f32)
    deg = _deg_kernel(dst.reshape(NW, PT // DEG_C, DEG_C), zeros_w, ones_c)

    src_ch = src.reshape(NW, CH, CL)
    dst_ch = dst.reshape(NW, CH, CL)
    dis, g1 = _tc1(xp, W1, deg)
    S1 = _scat_kernel(src_ch, dst_ch, g1, zeros_w)
    g2 = _tc2(dis, g1, S1, b1.reshape(1, 4), W2)
    S2 = _scat_kernel(src_ch, dst_ch, g2, zeros_w)
    out = _tc3(dis, g2, S2, b2.reshape(1, 2))
    return out[:N]


# speedup vs baseline: 67.8463x; 1.6665x over previous
import functools

import jax
import jax.numpy as jnp
from jax import lax
from jax.experimental import pallas as pl
from jax.experimental.pallas import tpu as pltpu
from jax.experimental.pallas import tpu_sc as plsc

N = 10000
E = 320000
NC, NS = 2, 16
NW = NC * NS
PT = E // NW
W = 8
CH = 10
CL = PT // CH
DEG_C = 2000

f32 = jnp.float32


def _mesh():
    return plsc.VectorSubcoreMesh(
        core_axis_name="c", subcore_axis_name="s", num_cores=NC, num_subcores=NS
    )


_SC_PARAMS = pltpu.CompilerParams(use_tc_tiling_on_sc=False)


@functools.partial(
    pl.kernel,
    mesh=_mesh(),
    out_type=jax.ShapeDtypeStruct((NC, N, W), f32),
    compiler_params=_SC_PARAMS,
    scratch_types=[
        pltpu.VMEM((PT // DEG_C, DEG_C), jnp.int32),
        pltpu.VMEM((DEG_C, W), f32),
        pltpu.VMEM_SHARED((N, W), f32),
        pltpu.SemaphoreType.DMA,
    ],
)
def _deg_kernel(dst_hbm, zeros_hbm, ones_hbm, out_hbm, idx_v, ones_v, acc_sh, sem):
    cid = lax.axis_index("c")
    sid = lax.axis_index("s")
    wid = sid * NC + cid

    @pl.when(sid == 0)
    def _():
        pltpu.sync_copy(zeros_hbm, acc_sh)

    pltpu.sync_copy(ones_hbm, ones_v)
    pltpu.sync_copy(dst_hbm.at[wid], idx_v)
    plsc.subcore_barrier()

    for k in range(PT // DEG_C):
        pltpu.async_copy(ones_v, acc_sh.at[idx_v.at[k]], sem, add=True)
    for k in range(PT // DEG_C):
        pltpu.make_async_copy(ones_v, acc_sh.at[idx_v.at[k]], sem).wait()

    plsc.subcore_barrier()

    @pl.when(sid == 0)
    def _():
        pltpu.sync_copy(acc_sh, out_hbm.at[cid])


@functools.partial(
    pl.kernel,
    mesh=_mesh(),
    out_type=jax.ShapeDtypeStruct((NC, N, W), f32),
    compiler_params=_SC_PARAMS,
    scratch_types=[
        pltpu.VMEM((CH, CL), jnp.int32),
        pltpu.VMEM((CH, CL), jnp.int32),
        pltpu.VMEM((2, CL, W), f32),
        pltpu.VMEM_SHARED((N, W), f32),
        pltpu.VMEM_SHARED((N, W), f32),
        pltpu.SemaphoreType.DMA,
        pltpu.SemaphoreType.DMA,
        pltpu.SemaphoreType.DMA,
        pltpu.SemaphoreType.DMA,
    ],
)
def _scat_kernel(src_hbm, dst_hbm, g_hbm, zeros_hbm, out_hbm,
                 src_v, dst_v, msg_v, g_sh, acc_sh, gs0, gs1, ss0, ss1):
    cid = lax.axis_index("c")
    sid = lax.axis_index("s")
    wid = sid * NC + cid
    gsem = (gs0, gs1)
    ssem = (ss0, ss1)

    @pl.when(sid == 0)
    def _():
        pltpu.sync_copy(zeros_hbm, acc_sh)

    @pl.when(sid == 1)
    def _():
        pltpu.sync_copy(g_hbm, g_sh)

    pltpu.sync_copy(src_hbm.at[wid], src_v)
    pltpu.sync_copy(dst_hbm.at[wid], dst_v)
    plsc.subcore_barrier()

    def gather(c, b):
        pltpu.async_copy(g_sh.at[src_v.at[c]], msg_v.at[b], gsem[b])

    def scatter(c, b):
        pltpu.async_copy(msg_v.at[b], acc_sh.at[dst_v.at[c]], ssem[b], add=True)

    gather(0, 0)
    for c in range(CH):
        b = c % 2
        pltpu.make_async_copy(g_sh.at[src_v.at[c]], msg_v.at[b], gsem[b]).wait()
        scatter(c, b)
        if c + 1 < CH:
            nb = 1 - b
            if c >= 1:
                pltpu.make_async_copy(
                    msg_v.at[nb], acc_sh.at[dst_v.at[c - 1]], ssem[nb]).wait()
            gather(c + 1, nb)
    pltpu.make_async_copy(
        msg_v.at[(CH - 1) % 2], acc_sh.at[dst_v.at[CH - 1]],
        ssem[(CH - 1) % 2]).wait()
    pltpu.make_async_copy(
        msg_v.at[(CH - 2) % 2], acc_sh.at[dst_v.at[CH - 2]],
        ssem[(CH - 2) % 2]).wait()

    plsc.subcore_barrier()

    @pl.when(sid == 0)
    def _():
        pltpu.sync_copy(acc_sh, out_hbm.at[cid])


def _tc1(x, W1, deg):
    def body(x_ref, w_ref, deg_ref, dis_ref, g_ref):
        h = jnp.dot(x_ref[...], w_ref[...], preferred_element_type=f32)
        dis = lax.rsqrt(deg_ref[0][:, :1] + deg_ref[1][:, :1] + 1.0)
        dis_ref[...] = dis
        g_ref[...] = jnp.concatenate([h * dis, jnp.zeros((N, W - 4), f32)], axis=1)

    return pl.pallas_call(
        body,
        out_shape=(
            jax.ShapeDtypeStruct((N, 1), f32),
            jax.ShapeDtypeStruct((N, W), f32),
        ),
    )(x, W1, deg)


def _tc2(dis, g1, S1, b1, W2):
    def body(dis_ref, g_ref, s_ref, b_ref, w_ref, g2_ref):
        acc = (g_ref[:, :4] + s_ref[0][:, :4] + s_ref[1][:, :4])
        act = jnp.maximum(dis_ref[...] * acc + b_ref[...], 0.0)
        h2 = jnp.dot(act, w_ref[...], preferred_element_type=f32)
        g2_ref[...] = jnp.concatenate(
            [h2 * dis_ref[...], jnp.zeros((N, W - 2), f32)], axis=1)

    return pl.pallas_call(
        body,
        out_shape=jax.ShapeDtypeStruct((N, W), f32),
    )(dis, g1, S1, b1, W2)


def _tc3(dis, g2, S2, b2):
    def body(dis_ref, g_ref, s_ref, b_ref, out_ref):
        acc = g_ref[:, :2] + s_ref[0][:, :2] + s_ref[1][:, :2]
        out_ref[...] = dis_ref[...] * acc + b_ref[...]

    return pl.pallas_call(
        body,
        out_shape=jax.ShapeDtypeStruct((N, 2), f32),
    )(dis, g2, S2, b2)


def kernel(x, edge_index, W1, b1, W2, b2):
    ei = edge_index.astype(jnp.int32)
    src_ch = ei[0].reshape(NW, CH, CL)
    dst_ch = ei[1].reshape(NW, CH, CL)
    dst_deg = ei[1].reshape(NW, PT // DEG_C, DEG_C)

    zeros_w = jnp.zeros((N, W), f32)
    ones_c = jnp.ones((DEG_C, W), f32)
    deg = _deg_kernel(dst_deg, zeros_w, ones_c)

    dis, g1 = _tc1(x, W1, deg)
    S1 = _scat_kernel(src_ch, dst_ch, g1, zeros_w)
    g2 = _tc2(dis, g1, S1, b1.reshape(1, 4), W2)
    S2 = _scat_kernel(src_ch, dst_ch, g2, zeros_w)
    out = _tc3(dis, g2, S2, b2.reshape(1, 2))
    return out
